# baseline (speedup 1.0000x reference)
import functools

import jax
import jax.numpy as jnp
from jax import lax
from jax.experimental import pallas as pl
from jax.experimental.pallas import tpu as pltpu
from jax.experimental.pallas import tpu_sc as plsc

_INFO = plsc.get_sparse_core_info()
_NC = _INFO.num_cores
_NS = _INFO.num_subcores
_NW = _NC * _NS
_CHUNK = 128
_GPB = 2
_BROWS = _CHUNK * _GPB
_NBUF = 3


@functools.lru_cache(maxsize=None)
def _make_gather(n_rows: int, d: int, chunks_per_w: int):
    rows_per_w = n_rows // _NW
    pairs = chunks_per_w // _GPB
    ngroups = pairs // _NBUF
    rem = pairs - ngroups * _NBUF
    mesh = plsc.VectorSubcoreMesh(core_axis_name="c", subcore_axis_name="s")

    @functools.partial(
        pl.kernel,
        mesh=mesh,
        out_type=jax.ShapeDtypeStruct((n_rows, d), jnp.float32),
        scratch_types=[
            pltpu.VMEM((chunks_per_w, _CHUNK), jnp.int32),
            pltpu.VMEM((_NBUF, _BROWS, d), jnp.float32),
        ]
        + [pltpu.SemaphoreType.DMA] * (2 * _NBUF),
    )
    def gather_kernel(table_hbm, idx_hbm, out_hbm, idx_v, bufs, *sems):
        gsems, ssems = sems[:_NBUF], sems[_NBUF:]
        wid = lax.axis_index("s") * _NC + lax.axis_index("c")
        base = wid * rows_per_w
        pltpu.sync_copy(idx_hbm.at[wid], idx_v)

        def gstart(p, b):
            for h in range(_GPB):
                pltpu.async_copy(
                    table_hbm.at[idx_v.at[p * _GPB + h]],
                    bufs.at[b].at[pl.ds(h * _CHUNK, _CHUNK)],
                    gsems[b],
                )

        def gwait(p, b):
            for h in range(_GPB):
                pltpu.make_async_copy(
                    table_hbm.at[idx_v.at[p * _GPB + h]],
                    bufs.at[b].at[pl.ds(h * _CHUNK, _CHUNK)],
                    gsems[b],
                ).wait()

        def sstart(p, b):
            pltpu.async_copy(
                bufs.at[b], out_hbm.at[pl.ds(base + p * _BROWS, _BROWS)], ssems[b]
            )

        def swait(p, b):
            pltpu.make_async_copy(
                bufs.at[b], out_hbm.at[pl.ds(base + p * _BROWS, _BROWS)], ssems[b]
            ).wait()

        for b in range(_NBUF):
            gstart(b, b)

        def body(g, carry):
            p0 = g * _NBUF
            for b in range(_NBUF):
                gwait(p0 + b, b)
                sstart(p0 + b, b)
            for b in range(_NBUF):
                swait(p0 + b, b)
                gstart(p0 + _NBUF + b, b)
            return carry

        lax.fori_loop(0, ngroups - 1, body, 0)

        p0 = (ngroups - 1) * _NBUF
        for b in range(_NBUF):
            gwait(p0 + b, b)
            sstart(p0 + b, b)
        for r in range(rem):
            swait(p0 + r, r)
            gstart(p0 + _NBUF + r, r)
        for r in range(rem):
            p = p0 + _NBUF + r
            gwait(p, r)
            sstart(p, r)
        for b in range(rem, _NBUF):
            swait(p0 + b, b)
        for r in range(rem):
            swait(p0 + _NBUF + r, r)

    return gather_kernel


def kernel(W, indices):
    b, h = indices.shape
    v, d = W.shape
    n_rows = b * h
    assert n_rows % (_NW * _CHUNK * _GPB) == 0
    chunks_per_w = n_rows // (_NW * _CHUNK)
    idx3 = indices.reshape(_NW, chunks_per_w, _CHUNK)
    out = _make_gather(n_rows, d, chunks_per_w)(W, idx3)
    return out.reshape(b, h, d)
